# baseline (speedup 1.0000x reference)
import functools

import jax
import jax.numpy as jnp
from jax import lax
from jax.experimental import pallas as pl
from jax.experimental.pallas import tpu as pltpu
from jax.experimental.pallas import tpu_sc as plsc

TRUST_W = 0.4
SIM_W = 0.4
STALE_W = 0.2

TM = 128
NW = 32
NTEXP = 48


def _prefix_rows(a):
    n = a.shape[0]
    s = 1
    while s < n:
        a = a + jnp.concatenate(
            [jnp.zeros((s, a.shape[1]), a.dtype), a[:-s]], axis=0)
        s *= 2
    return a


def _route_kernel(x_ref, fn_ref, trust_ref, stale_ref,
                  pos0_ref, pos1_ref, texp_ref):
    x = x_ref[...]
    fn = fn_ref[...]
    E = fn.shape[0]
    eps = 1e-8
    xnorm = jnp.maximum(jnp.sqrt(jnp.sum(x * x, axis=1, keepdims=True)), eps)
    fnorm = jnp.maximum(jnp.sqrt(jnp.sum(fn * fn, axis=1, keepdims=True)), eps)
    s = lax.dot_general(x / xnorm, fn / fnorm, (((1,), (1,)), ((), ())),
                        preferred_element_type=jnp.float32,
                        precision=lax.Precision.HIGHEST)
    sim = (s + 1.0) * 0.5
    stale_score = jnp.maximum(0.0, 1.0 - stale_ref[...])
    scores = TRUST_W * trust_ref[...] + SIM_W * sim + STALE_W * stale_score

    eidx = lax.broadcasted_iota(jnp.int32, scores.shape, 1)
    nbetter = jnp.zeros(scores.shape, jnp.int32)
    for j in range(E):
        colj = scores[:, j:j + 1]
        nbetter += ((scores < colj) | ((scores == colj) & (eidx > j))
                    ).astype(jnp.int32)
    oh0 = (nbetter == 0).astype(jnp.int32)
    oh1 = (nbetter == 1).astype(jnp.int32)

    c0 = _prefix_rows(oh0)
    c1 = _prefix_rows(oh1)
    rank0 = c0 - oh0
    rank1 = c1 - oh1
    n0 = c0[-1:, :]
    n = n0 + c1[-1:, :]
    padded = ((n + (TM - 1)) // TM) * TM
    coff = padded
    s2 = 1
    while s2 < E:
        coff = coff + jnp.concatenate(
            [jnp.zeros((1, s2), coff.dtype), coff[:, :-s2]], axis=1)
        s2 *= 2
    off = coff - padded

    pos0_ref[...] = jnp.sum(oh0 * (off + rank0), axis=1, keepdims=True)
    pos1_ref[...] = jnp.sum(oh1 * (off + n0 + rank1), axis=1, keepdims=True)

    tt = lax.broadcasted_iota(jnp.int32, (NTEXP, E), 0) * TM
    inr = (tt >= off) & (tt < off + padded)
    texp_ref[...] = jnp.sum(
        jnp.where(inr, lax.broadcasted_iota(jnp.int32, (NTEXP, E), 1), 0),
        axis=1, keepdims=True)


def _dispatch_body(x_hbm, p0_hbm, p1_hbm, xs_hbm, rows_v, p0_v, p1_v, sem):
    nc = 2
    wid = lax.axis_index("s") * nc + lax.axis_index("c")
    tpw = rows_v.shape[0]
    base = wid * tpw
    pltpu.sync_copy(x_hbm.at[pl.ds(base, tpw)], rows_v)
    pltpu.sync_copy(p0_hbm.at[pl.ds(base, tpw)], p0_v)
    pltpu.sync_copy(p1_hbm.at[pl.ds(base, tpw)], p1_v)
    pltpu.async_copy(rows_v, xs_hbm.at[p0_v], sem).wait()
    pltpu.async_copy(rows_v, xs_hbm.at[p1_v], sem).wait()


def _gmm_kernel(texp_ref, xs_ref, w_ref, b_ref, invk_ref, ys_ref):
    y = lax.dot_general(xs_ref[...], w_ref[0], (((1,), (1,)), ((), ())),
                        preferred_element_type=jnp.float32)
    ys_ref[...] = invk_ref[0, 0] * (y + b_ref[0])


def _combine_body(ys_hbm, p0_hbm, p1_hbm, out_hbm, a_v, b_v, p0_v, p1_v, sem):
    nc = 2
    wid = lax.axis_index("s") * nc + lax.axis_index("c")
    ch = a_v.shape[0]
    d = a_v.shape[1]
    tpw = out_hbm.shape[0] // NW
    base = wid * tpw
    for c in range(tpw // ch):
        cb = base + c * ch
        pltpu.sync_copy(p0_hbm.at[pl.ds(cb, ch)], p0_v)
        pltpu.sync_copy(p1_hbm.at[pl.ds(cb, ch)], p1_v)
        pltpu.async_copy(ys_hbm.at[p0_v], a_v, sem).wait()
        pltpu.async_copy(ys_hbm.at[p1_v], b_v, sem).wait()

        def _row(r, carry):
            for q in range(d // 16):
                sl = pl.ds(q * 16, 16)
                a_v[r, sl] = a_v[r, sl] + b_v[r, sl]
            return carry

        lax.fori_loop(0, ch, _row, 0)
        pltpu.sync_copy(a_v, out_hbm.at[pl.ds(cb, ch)])


def kernel(x, trust_scores, representative_features, staleness, expert_W,
           expert_b, k):
    B, D = x.shape
    E, C, _ = expert_W.shape
    P = B * 2 + E * TM
    NT = P // TM
    inv_k = jnp.asarray(1.0 / k, dtype=jnp.float32).reshape(1, 1)
    trust2 = trust_scores.reshape(1, E)
    stale2 = staleness.reshape(1, E)

    pos0, pos1, texp = pl.pallas_call(
        _route_kernel,
        grid=(1,),
        in_specs=[
            pl.BlockSpec((B, D), lambda i: (0, 0)),
            pl.BlockSpec((E, D), lambda i: (0, 0)),
            pl.BlockSpec((1, E), lambda i: (0, 0)),
            pl.BlockSpec((1, E), lambda i: (0, 0)),
        ],
        out_specs=[
            pl.BlockSpec((B, 1), lambda i: (0, 0)),
            pl.BlockSpec((B, 1), lambda i: (0, 0)),
            pl.BlockSpec((NTEXP, 1), lambda i: (0, 0)),
        ],
        out_shape=[
            jax.ShapeDtypeStruct((B, 1), jnp.int32),
            jax.ShapeDtypeStruct((B, 1), jnp.int32),
            jax.ShapeDtypeStruct((NTEXP, 1), jnp.int32),
        ],
    )(x, representative_features, trust2, stale2)
    pos0 = pos0.reshape(B)
    pos1 = pos1.reshape(B)
    texp = texp.reshape(NTEXP)

    mesh = plsc.VectorSubcoreMesh(core_axis_name="c", subcore_axis_name="s")
    tpw = B // NW

    xs = pl.kernel(
        _dispatch_body,
        out_type=jax.ShapeDtypeStruct((P, D), jnp.float32),
        mesh=mesh,
        scratch_types=[
            pltpu.VMEM((tpw, D), jnp.float32),
            pltpu.VMEM((tpw,), jnp.int32),
            pltpu.VMEM((tpw,), jnp.int32),
            pltpu.SemaphoreType.DMA,
        ],
    )(x, pos0, pos1)

    ys = pl.pallas_call(
        _gmm_kernel,
        grid_spec=pltpu.PrefetchScalarGridSpec(
            num_scalar_prefetch=1,
            grid=(NT,),
            in_specs=[
                pl.BlockSpec((TM, D), lambda t, texp: (t, 0)),
                pl.BlockSpec((1, C, D), lambda t, texp: (texp[t], 0, 0)),
                pl.BlockSpec((1, 1, C), lambda t, texp: (texp[t], 0, 0)),
                pl.BlockSpec((1, 1), lambda t, texp: (0, 0)),
            ],
            out_specs=pl.BlockSpec((TM, C), lambda t, texp: (t, 0)),
        ),
        out_shape=jax.ShapeDtypeStruct((P, C), jnp.float32),
    )(texp, xs, expert_W, expert_b.reshape(E, 1, C), inv_k)

    out = pl.kernel(
        _combine_body,
        out_type=jax.ShapeDtypeStruct((B, C), jnp.float32),
        mesh=mesh,
        scratch_types=[
            pltpu.VMEM((tpw // 2, C), jnp.float32),
            pltpu.VMEM((tpw // 2, C), jnp.float32),
            pltpu.VMEM((tpw // 2,), jnp.int32),
            pltpu.VMEM((tpw // 2,), jnp.int32),
            pltpu.SemaphoreType.DMA,
        ],
    )(ys, pos0, pos1)
    return out


# speedup vs baseline: 1.8604x; 1.8604x over previous
import functools

import jax
import jax.numpy as jnp
from jax.experimental import pallas as pl
from jax.experimental.pallas import tpu as pltpu

TRUST_W = 0.4
SIM_W = 0.4
STALE_W = 0.2


def _fused_kernel(x_ref, fn_ref, trust_ref, stale_ref, w_ref, b_ref, invk_ref,
                  out_ref, wsel_ref):
    e = pl.program_id(1)

    @pl.when(e == 0)
    def _scores():
        x = x_ref[...]
        fn = fn_ref[...]
        eps = 1e-8
        xnorm = jnp.maximum(jnp.sqrt(jnp.sum(x * x, axis=1, keepdims=True)),
                            eps)
        fnorm = jnp.maximum(jnp.sqrt(jnp.sum(fn * fn, axis=1, keepdims=True)),
                            eps)
        xn = x / xnorm
        fnn = fn / fnorm
        s = jax.lax.dot_general(xn, fnn, (((1,), (1,)), ((), ())),
                                preferred_element_type=jnp.float32,
                                precision=jax.lax.Precision.HIGHEST)
        sim = (s + 1.0) * 0.5
        stale_score = jnp.maximum(0.0, 1.0 - stale_ref[...])
        scores = (TRUST_W * trust_ref[...] + SIM_W * sim
                  + STALE_W * stale_score)
        eidx = jax.lax.broadcasted_iota(jnp.int32, scores.shape, 1)
        nbetter = jnp.zeros(scores.shape, jnp.int32)
        for j in range(scores.shape[1]):
            colj = scores[:, j:j + 1]
            nbetter += ((scores < colj) | ((scores == colj) & (eidx > j))
                        ).astype(jnp.int32)
        wsel_ref[...] = jnp.where(nbetter < 2, invk_ref[0, 0], 0.0)

    eidx2 = jax.lax.broadcasted_iota(jnp.int32, wsel_ref.shape, 1)
    w = jnp.sum(jnp.where(eidx2 == e, wsel_ref[...], 0.0), axis=1,
                keepdims=True)
    y = jax.lax.dot_general(x_ref[...], w_ref[0], (((1,), (1,)), ((), ())),
                            preferred_element_type=jnp.float32)
    contrib = w * (y + b_ref[0])

    @pl.when(e == 0)
    def _():
        out_ref[...] = contrib

    @pl.when(e != 0)
    def _():
        out_ref[...] += contrib


def kernel(x, trust_scores, representative_features, staleness, expert_W,
           expert_b, k):
    B, D = x.shape
    E, C, _ = expert_W.shape
    BT = 2048
    inv_k = jnp.asarray(1.0 / k, dtype=jnp.float32).reshape(1, 1)
    trust2 = trust_scores.reshape(1, E)
    stale2 = staleness.reshape(1, E)

    grid = (B // BT, E)
    out = pl.pallas_call(
        _fused_kernel,
        grid=grid,
        in_specs=[
            pl.BlockSpec((BT, D), lambda b, e: (b, 0)),
            pl.BlockSpec((E, D), lambda b, e: (0, 0)),
            pl.BlockSpec((1, E), lambda b, e: (0, 0)),
            pl.BlockSpec((1, E), lambda b, e: (0, 0)),
            pl.BlockSpec((1, C, D), lambda b, e: (e, 0, 0)),
            pl.BlockSpec((1, 1, C), lambda b, e: (e, 0, 0)),
            pl.BlockSpec((1, 1), lambda b, e: (0, 0)),
        ],
        out_specs=pl.BlockSpec((BT, C), lambda b, e: (b, 0)),
        out_shape=jax.ShapeDtypeStruct((B, C), jnp.float32),
        scratch_shapes=[pltpu.VMEM((BT, E), jnp.float32)],
    )(x, representative_features, trust2, stale2, expert_W,
      expert_b.reshape(E, 1, C), inv_k)
    return out


# speedup vs baseline: 2.0212x; 1.0864x over previous
import functools

import jax
import jax.numpy as jnp
from jax.experimental import pallas as pl
from jax.experimental.pallas import tpu as pltpu

TRUST_W = 0.4
SIM_W = 0.4
STALE_W = 0.2


def _fused_kernel(x_ref, fn_ref, trust_ref, stale_ref, w_ref, b_ref, invk_ref,
                  out_ref, wsel_ref):
    e = pl.program_id(1)

    @pl.when(e == 0)
    def _scores():
        x = x_ref[...]
        fn = fn_ref[...]
        eps = 1e-8
        xnorm = jnp.maximum(jnp.sqrt(jnp.sum(x * x, axis=1, keepdims=True)),
                            eps)
        fnorm = jnp.maximum(jnp.sqrt(jnp.sum(fn * fn, axis=1, keepdims=True)),
                            eps)
        xn = x / xnorm
        fnn = fn / fnorm
        s = jax.lax.dot_general(xn, fnn, (((1,), (1,)), ((), ())),
                                preferred_element_type=jnp.float32)
        sim = (s + 1.0) * 0.5
        stale_score = jnp.maximum(0.0, 1.0 - stale_ref[...])
        scores = (TRUST_W * trust_ref[...] + SIM_W * sim
                  + STALE_W * stale_score)
        eidx = jax.lax.broadcasted_iota(jnp.int32, scores.shape, 1)
        nbetter = jnp.zeros(scores.shape, jnp.int32)
        for j in range(scores.shape[1]):
            colj = scores[:, j:j + 1]
            nbetter += ((scores < colj) | ((scores == colj) & (eidx > j))
                        ).astype(jnp.int32)
        wsel_ref[...] = jnp.where(nbetter < 2, invk_ref[0, 0], 0.0)

    eidx2 = jax.lax.broadcasted_iota(jnp.int32, wsel_ref.shape, 1)
    w = jnp.sum(jnp.where(eidx2 == e, wsel_ref[...], 0.0), axis=1,
                keepdims=True)
    y = jax.lax.dot_general(x_ref[...], w_ref[0], (((1,), (1,)), ((), ())),
                            preferred_element_type=jnp.float32)
    contrib = w * (y + b_ref[0])

    @pl.when(e == 0)
    def _():
        out_ref[...] = contrib

    @pl.when(e != 0)
    def _():
        out_ref[...] += contrib


def kernel(x, trust_scores, representative_features, staleness, expert_W,
           expert_b, k):
    B, D = x.shape
    E, C, _ = expert_W.shape
    BT = 2048
    inv_k = jnp.asarray(1.0 / k, dtype=jnp.float32).reshape(1, 1)
    trust2 = trust_scores.reshape(1, E)
    stale2 = staleness.reshape(1, E)

    grid = (B // BT, E)
    out = pl.pallas_call(
        _fused_kernel,
        grid=grid,
        in_specs=[
            pl.BlockSpec((BT, D), lambda b, e: (b, 0)),
            pl.BlockSpec((E, D), lambda b, e: (0, 0)),
            pl.BlockSpec((1, E), lambda b, e: (0, 0)),
            pl.BlockSpec((1, E), lambda b, e: (0, 0)),
            pl.BlockSpec((1, C, D), lambda b, e: (e, 0, 0)),
            pl.BlockSpec((1, 1, C), lambda b, e: (e, 0, 0)),
            pl.BlockSpec((1, 1), lambda b, e: (0, 0)),
        ],
        out_specs=pl.BlockSpec((BT, C), lambda b, e: (b, 0)),
        out_shape=jax.ShapeDtypeStruct((B, C), jnp.float32),
        scratch_shapes=[pltpu.VMEM((BT, E), jnp.float32)],
    )(x, representative_features, trust2, stale2, expert_W,
      expert_b.reshape(E, 1, C), inv_k)
    return out
